# baseline (speedup 1.0000x reference)
import jax
import jax.numpy as jnp
from jax import lax
from jax.experimental import pallas as pl
from jax.experimental.pallas import tpu as pltpu
from jax.experimental.pallas import tpu_sc as plsc

B = 16384
D = 64
V = 1000001
NC = 2
NS = 16
NW = NC * NS
QPW = B // NW
BLK = 32768
NBLK = (V + BLK - 1) // BLK
VP = NBLK * BLK


def _k2_body(sidx_ref, t_ref, t2_hbm, d0_ref, d1_ref, n_ref, isn_ref,
             st_v, buf_v, d0_v, d1_v, n_v, sem, osem):
    i = pl.program_id(0)

    @pl.when(i == 0)
    def _support():
        for j in range(10):
            v = sidx_ref[j]
            base = pl.multiple_of((v // 128) * 128, 128)
            pltpu.make_async_copy(
                t2_hbm.at[:, pl.ds(base, 128)], buf_v.at[j], sem).start()
        for j in range(10):
            pltpu.make_async_copy(
                t2_hbm.at[:, pl.ds(0, 128)], buf_v.at[j], sem).wait()
        lane = lax.broadcasted_iota(jnp.int32, (1, 128), 1)
        cols = []
        for j in range(10):
            m = (lane == (sidx_ref[j] % 128)).astype(jnp.float32)
            cols.append(jnp.sum(buf_v[j] * m, axis=1, keepdims=True))
        s0 = (cols[0] + cols[2] + cols[4] + cols[6] + cols[8]) * 0.2
        s1 = (cols[1] + cols[3] + cols[5] + cols[7] + cols[9]) * 0.2
        st_v[...] = jnp.concatenate([s0, s1], axis=1)
        sn2 = jnp.sum(s0 * s0) + jnp.sum(s1 * s1)
        inv_sn = 1.0 / jnp.maximum(jnp.sqrt(sn2), 1e-8)
        isn_ref[...] = jnp.full((16,), inv_sn, jnp.float32)

    t = t_ref[...]
    d01 = jax.lax.dot_general(st_v[...], t, (((0,), (0,)), ((), ())),
                              preferred_element_type=jnp.float32)
    slot = lax.rem(i, 2)
    d0_v[slot] = d01[0, :]
    d1_v[slot] = d01[1, :]
    n_v[slot] = jnp.sum(t * t, axis=0)

    def _cps(step, s):
        off = pl.ds(step * BLK, BLK)
        return [pltpu.make_async_copy(d0_v.at[s], d0_ref.at[off], osem),
                pltpu.make_async_copy(d1_v.at[s], d1_ref.at[off], osem),
                pltpu.make_async_copy(n_v.at[s], n_ref.at[off], osem)]

    @pl.when(i >= 2)
    def _drain_prev():
        for c in _cps(i - 2, slot):
            c.wait()

    for c in _cps(i, slot):
        c.start()

    @pl.when(i == NBLK - 1)
    def _drain_tail():
        for c in _cps(NBLK - 2, 1 - slot) + _cps(NBLK - 1, slot):
            c.wait()


_k2 = pl.pallas_call(
    _k2_body,
    grid=(NBLK,),
    in_specs=[
        pl.BlockSpec(memory_space=pltpu.SMEM),
        pl.BlockSpec((D, BLK), lambda i: (0, i)),
        pl.BlockSpec(memory_space=pl.ANY),
    ],
    out_specs=[
        pl.BlockSpec(memory_space=pltpu.MemorySpace.HBM),
        pl.BlockSpec(memory_space=pltpu.MemorySpace.HBM),
        pl.BlockSpec(memory_space=pltpu.MemorySpace.HBM),
        pl.BlockSpec((16,), lambda i: (0,)),
    ],
    scratch_shapes=[
        pltpu.VMEM((D, 2), jnp.float32),
        pltpu.VMEM((10, D, 128), jnp.float32),
        pltpu.VMEM((2, BLK), jnp.float32),
        pltpu.VMEM((2, BLK), jnp.float32),
        pltpu.VMEM((2, BLK), jnp.float32),
        pltpu.SemaphoreType.DMA,
        pltpu.SemaphoreType.DMA,
    ],
    out_shape=[
        jax.ShapeDtypeStruct((VP,), jnp.float32),
        jax.ShapeDtypeStruct((VP,), jnp.float32),
        jax.ShapeDtypeStruct((VP,), jnp.float32),
        jax.ShapeDtypeStruct((16,), jnp.float32),
    ],
)


def _rsqrt16(x):
    i = lax.bitcast_convert_type(x, jnp.int32)
    i = jnp.int32(0x5F3759DF) - (i >> 1)
    y = lax.bitcast_convert_type(i, jnp.float32)
    for _ in range(3):
        y = y * (1.5 - 0.5 * x * y * y)
    return y


def _k3_body(qt_hbm, d0_hbm, d1_hbm, n_hbm, isn_hbm, out_hbm,
             ia_v, ib_v, ga_v, gb_v, na_v, nb_v, isn_v, out_v, sem):
    wid = lax.axis_index("s") * NC + lax.axis_index("c")
    pltpu.sync_copy(qt_hbm.at[0, pl.ds(wid * QPW, QPW)], ia_v)
    pltpu.sync_copy(qt_hbm.at[1, pl.ds(wid * QPW, QPW)], ib_v)
    pltpu.sync_copy(isn_hbm, isn_v)
    copies = []
    for r in range(QPW // 128):
        sl = pl.ds(r * 128, 128)
        copies.append(pltpu.async_copy(d0_hbm.at[ia_v.at[sl]], ga_v.at[sl], sem))
        copies.append(pltpu.async_copy(d1_hbm.at[ib_v.at[sl]], gb_v.at[sl], sem))
        copies.append(pltpu.async_copy(n_hbm.at[ia_v.at[sl]], na_v.at[sl], sem))
        copies.append(pltpu.async_copy(n_hbm.at[ib_v.at[sl]], nb_v.at[sl], sem))
    for c in copies:
        c.wait()
    inv_sn = isn_v[...]
    for g in range(QPW // 16):
        sl = pl.ds(g * 16, 16)
        y = _rsqrt16(jnp.maximum(na_v[sl] + nb_v[sl], 1e-16))
        out_v[sl] = (ga_v[sl] + gb_v[sl]) * y * inv_sn
    pltpu.sync_copy(out_v, out_hbm.at[pl.ds(wid * QPW, QPW)])


_k3 = pl.kernel(
    _k3_body,
    out_type=jax.ShapeDtypeStruct((B,), jnp.float32),
    mesh=plsc.VectorSubcoreMesh(core_axis_name="c", subcore_axis_name="s"),
    compiler_params=pltpu.CompilerParams(
        needs_layout_passes=False, use_tc_tiling_on_sc=True),
    scratch_types=[
        pltpu.VMEM((QPW,), jnp.int32),
        pltpu.VMEM((QPW,), jnp.int32),
        pltpu.VMEM((QPW,), jnp.float32),
        pltpu.VMEM((QPW,), jnp.float32),
        pltpu.VMEM((QPW,), jnp.float32),
        pltpu.VMEM((QPW,), jnp.float32),
        pltpu.VMEM((16,), jnp.float32),
        pltpu.VMEM((QPW,), jnp.float32),
        pltpu.SemaphoreType.DMA,
    ],
)


def kernel(query, support, table):
    t2 = table.T
    qt = query.astype(jnp.int32).T
    sidx = jnp.pad(support.astype(jnp.int32).reshape(-1), (0, 6))
    d0, d1, n, isn = _k2(sidx, t2, t2)
    return _k3(qt, d0, d1, n, isn)


# speedup vs baseline: 1.0065x; 1.0065x over previous
import jax
import jax.numpy as jnp
from jax import lax
from jax.experimental import pallas as pl
from jax.experimental.pallas import tpu as pltpu
from jax.experimental.pallas import tpu_sc as plsc

B = 16384
D = 64
V = 1000001
NC = 2
NS = 16
NW = NC * NS
QPW = B // NW
BLK = 32768
NBLK = (V + BLK - 1) // BLK


def _k2_body(sidx_ref, t_ref, t2_hbm, d0_ref, d1_ref, n_ref, isn_ref,
             st_v, buf_v, sem):
    i = pl.program_id(0)

    @pl.when(i == 0)
    def _support():
        for j in range(10):
            v = sidx_ref[j]
            base = pl.multiple_of((v // 128) * 128, 128)
            pltpu.make_async_copy(
                t2_hbm.at[:, pl.ds(base, 128)], buf_v.at[j], sem).start()
        for j in range(10):
            pltpu.make_async_copy(
                t2_hbm.at[:, pl.ds(0, 128)], buf_v.at[j], sem).wait()
        lane = lax.broadcasted_iota(jnp.int32, (1, 128), 1)
        cols = []
        for j in range(10):
            m = (lane == (sidx_ref[j] % 128)).astype(jnp.float32)
            cols.append(jnp.sum(buf_v[j] * m, axis=1, keepdims=True))
        s0 = (cols[0] + cols[2] + cols[4] + cols[6] + cols[8]) * 0.2
        s1 = (cols[1] + cols[3] + cols[5] + cols[7] + cols[9]) * 0.2
        st_v[...] = jnp.concatenate([s0, s1], axis=1)
        sn2 = jnp.sum(s0 * s0) + jnp.sum(s1 * s1)
        inv_sn = 1.0 / jnp.maximum(jnp.sqrt(sn2), 1e-8)
        isn_ref[...] = jnp.full((16,), inv_sn, jnp.float32)

    t = t_ref[...]
    d01 = jax.lax.dot_general(st_v[...], t, (((0,), (0,)), ((), ())),
                              preferred_element_type=jnp.float32)
    d0_ref[...] = d01[0, :]
    d1_ref[...] = d01[1, :]
    n_ref[...] = jnp.sum(t * t, axis=0)


_k2 = pl.pallas_call(
    _k2_body,
    grid=(NBLK,),
    in_specs=[
        pl.BlockSpec(memory_space=pltpu.SMEM),
        pl.BlockSpec((D, BLK), lambda i: (0, i)),
        pl.BlockSpec(memory_space=pl.ANY),
    ],
    out_specs=[
        pl.BlockSpec((BLK,), lambda i: (i,)),
        pl.BlockSpec((BLK,), lambda i: (i,)),
        pl.BlockSpec((BLK,), lambda i: (i,)),
        pl.BlockSpec((16,), lambda i: (0,)),
    ],
    scratch_shapes=[
        pltpu.VMEM((D, 2), jnp.float32),
        pltpu.VMEM((10, D, 128), jnp.float32),
        pltpu.SemaphoreType.DMA,
    ],
    out_shape=[
        jax.ShapeDtypeStruct((V,), jnp.float32),
        jax.ShapeDtypeStruct((V,), jnp.float32),
        jax.ShapeDtypeStruct((V,), jnp.float32),
        jax.ShapeDtypeStruct((16,), jnp.float32),
    ],
)


def _rsqrt16(x):
    i = lax.bitcast_convert_type(x, jnp.int32)
    i = jnp.int32(0x5F3759DF) - (i >> 1)
    y = lax.bitcast_convert_type(i, jnp.float32)
    for _ in range(3):
        y = y * (1.5 - 0.5 * x * y * y)
    return y


def _k3_body(qt_hbm, d0_hbm, d1_hbm, n_hbm, isn_hbm, out_hbm,
             ia_v, ib_v, ga_v, gb_v, na_v, nb_v, isn_v, out_v, sem):
    wid = lax.axis_index("s") * NC + lax.axis_index("c")
    pltpu.sync_copy(qt_hbm.at[0, pl.ds(wid * QPW, QPW)], ia_v)
    pltpu.sync_copy(qt_hbm.at[1, pl.ds(wid * QPW, QPW)], ib_v)
    pltpu.sync_copy(isn_hbm, isn_v)
    copies = []
    for r in range(QPW // 128):
        sl = pl.ds(r * 128, 128)
        copies.append(pltpu.async_copy(d0_hbm.at[ia_v.at[sl]], ga_v.at[sl], sem))
        copies.append(pltpu.async_copy(d1_hbm.at[ib_v.at[sl]], gb_v.at[sl], sem))
        copies.append(pltpu.async_copy(n_hbm.at[ia_v.at[sl]], na_v.at[sl], sem))
        copies.append(pltpu.async_copy(n_hbm.at[ib_v.at[sl]], nb_v.at[sl], sem))
    for c in copies:
        c.wait()
    inv_sn = isn_v[...]
    for g in range(QPW // 16):
        sl = pl.ds(g * 16, 16)
        y = _rsqrt16(jnp.maximum(na_v[sl] + nb_v[sl], 1e-16))
        out_v[sl] = (ga_v[sl] + gb_v[sl]) * y * inv_sn
    pltpu.sync_copy(out_v, out_hbm.at[pl.ds(wid * QPW, QPW)])


_k3 = pl.kernel(
    _k3_body,
    out_type=jax.ShapeDtypeStruct((B,), jnp.float32),
    mesh=plsc.VectorSubcoreMesh(core_axis_name="c", subcore_axis_name="s"),
    compiler_params=pltpu.CompilerParams(
        needs_layout_passes=False, use_tc_tiling_on_sc=True),
    scratch_types=[
        pltpu.VMEM((QPW,), jnp.int32),
        pltpu.VMEM((QPW,), jnp.int32),
        pltpu.VMEM((QPW,), jnp.float32),
        pltpu.VMEM((QPW,), jnp.float32),
        pltpu.VMEM((QPW,), jnp.float32),
        pltpu.VMEM((QPW,), jnp.float32),
        pltpu.VMEM((16,), jnp.float32),
        pltpu.VMEM((QPW,), jnp.float32),
        pltpu.SemaphoreType.DMA,
    ],
)


def kernel(query, support, table):
    t2 = table.T
    qt = query.astype(jnp.int32).T
    sidx = jnp.pad(support.astype(jnp.int32).reshape(-1), (0, 6))
    d0, d1, n, isn = _k2(sidx, t2, t2)
    return _k3(qt, d0, d1, n, isn)
